# Initial kernel scaffold; baseline (speedup 1.0000x reference)
import jax
import jax.numpy as jnp
from jax.experimental import pallas as pl


def kernel(x, u, edge_index, edge_w, loc, params):
    raise NotImplementedError("write your pallas kernel here")


# speedup vs baseline: 10.1758x; 10.1758x over previous
import functools

import numpy as np
import jax
import jax.numpy as jnp
from jax import lax
from jax.experimental import pallas as pl
from jax.experimental.pallas import tpu as pltpu
from jax.experimental.pallas import tpu_sc as plsc

B = 32; CITY = 1000; G = 16; E = 20000; TW = 24; FEAT = 8
X_EM = 32; LOC_EM = 8; DATE_EM = 8; EDGE_H = 16; GNN_H = 32; GNN_LAYER = 2
PRED = 6
N = B * CITY

_p = np.arange(G * (G - 1))
_i = _p // (G - 1)
_q = _p % (G - 1)
_j = _q + (_q >= _i)
S_II = (np.arange(G)[None, :] == _i[:, None]).astype(np.float32)
S_JJ = (np.arange(G)[None, :] == _j[:, None]).astype(np.float32)
_r = np.arange(G * G)
_ri, _rj = _r // G, _r % G
_tgt = _ri * (G - 1) + _rj - (_rj > _ri)
S_FULL = ((_p[None, :] == _tgt[:, None]) & (_ri != _rj)[:, None]).astype(np.float32)

_EC = 10000


def _edge_body(z_hbm, row_hbm, col_hbm, ew_hbm, wcol_hbm, acc_hbm, deg_hbm,
               zloc, accv, degv, rowv, colv, eww, wcolv):
    c = lax.axis_index("c")
    s = lax.axis_index("s")
    b = c * 16 + s
    ZW = CITY * GNN_H
    pltpu.sync_copy(z_hbm.at[pl.ds(b * ZW, ZW)], zloc)
    pltpu.sync_copy(wcol_hbm, wcolv)
    wlo = wcolv[pl.ds(0, 16)]
    whi = wcolv[pl.ds(16, 16)]
    zero16 = jnp.zeros((16,), jnp.float32)
    ones16 = jnp.ones((16,), jnp.float32)

    def zacc(i, carry):
        accv[pl.ds(i * 16, 16)] = zero16
        return carry

    lax.fori_loop(0, ZW // 16, zacc, 0)

    def zdeg(i, carry):
        degv[pl.ds(i * 16, 16)] = zero16
        return carry

    lax.fori_loop(0, CITY, zdeg, 0)

    for ch in range(E // _EC):
        eo = b * E + ch * _EC
        pltpu.sync_copy(row_hbm.at[pl.ds(eo, _EC)], rowv)
        pltpu.sync_copy(col_hbm.at[pl.ds(eo, _EC)], colv)
        pltpu.sync_copy(ew_hbm.at[pl.ds(eo, _EC)], eww)

        def ebody(g, carry):
            base = g * 16
            rv = rowv[pl.ds(base, 16)] * GNN_H
            cv = colv[pl.ds(base, 16)] * GNN_H
            wv = eww[pl.ds(base, 16)]
            for l in range(16):
                rb = rv[l]
                cb = cv[l]
                we = wv[l]
                v0 = zloc[pl.ds(rb, 16)]
                v1 = zloc[pl.ds(rb + 16, 16)]
                m0 = jnp.maximum(v0 + we * wlo, 0.0)
                m1 = jnp.maximum(v1 + we * whi, 0.0)
                plsc.addupdate(accv.at[pl.ds(cb, 16)], m0)
                plsc.addupdate(accv.at[pl.ds(cb + 16, 16)], m1)
                plsc.addupdate(degv.at[pl.ds(cb // 2, 16)], ones16)
            return carry

        lax.fori_loop(0, _EC // 16, ebody, 0)

    pltpu.sync_copy(accv, acc_hbm.at[pl.ds(b * ZW, ZW)])
    pltpu.sync_copy(degv, deg_hbm.at[pl.ds(b * CITY * 16, CITY * 16)])


@jax.jit
def _edge_pass(z, row, col, ew, wcol):
    mesh = plsc.VectorSubcoreMesh(core_axis_name="c", subcore_axis_name="s")
    f = functools.partial(
        pl.kernel,
        mesh=mesh,
        out_type=[
            jax.ShapeDtypeStruct((B * CITY * GNN_H,), jnp.float32),
            jax.ShapeDtypeStruct((B * CITY * 16,), jnp.float32),
        ],
        scratch_types=[
            pltpu.VMEM((CITY * GNN_H,), jnp.float32),
            pltpu.VMEM((CITY * GNN_H,), jnp.float32),
            pltpu.VMEM((CITY * 16,), jnp.float32),
            pltpu.VMEM((_EC,), jnp.int32),
            pltpu.VMEM((_EC,), jnp.int32),
            pltpu.VMEM((_EC,), jnp.float32),
            pltpu.VMEM((32,), jnp.float32),
        ],
    )(_edge_body)
    return f(z, row, col, ew, wcol)


def _lstm_encode(p, xr):
    Wih, Whh, bih, bhh = p
    Bn = xr.shape[0]
    h0 = jnp.zeros((Bn, X_EM), xr.dtype)
    c0 = jnp.zeros((Bn, X_EM), xr.dtype)

    def step(carry, xt):
        h, c = carry
        g = xt @ Wih.T + h @ Whh.T + bih + bhh
        i, f, gg, o = jnp.split(g, 4, axis=-1)
        i = jax.nn.sigmoid(i); f = jax.nn.sigmoid(f)
        gg = jnp.tanh(gg); o = jax.nn.sigmoid(o)
        c = f * c + i * gg
        h = o * jnp.tanh(c)
        return (h, c), None

    (h, c), _ = lax.scan(step, (h0, c0), jnp.swapaxes(xr, 0, 1))
    return h


def _group_layer(p, X, gew):
    m1W, m1b, m2W, m2b = p
    D = X.shape[-1]
    W1x, W1a = m1W[:, :D], m1W[:, D:]
    z = X @ W1x.T + m1b
    t = gew @ W1a.T
    t_full = jnp.einsum('rp,bpf->brf', jnp.asarray(S_FULL), t).reshape(B, G, G, GNN_H)
    msg = jax.nn.relu(z[:, :, None, :] + t_full)
    mean = (msg.sum(axis=1) - jax.nn.relu(z)) / (G - 1.0)
    out = jnp.concatenate([X, mean], axis=-1)
    return jax.nn.relu(out @ m2W.T + m2b)


def _global_layer(p, X, row, col, ew, inv_deg):
    m1W, m1b, m2W, m2b = p
    D = X.shape[-1]
    W1x, wcol = m1W[:, :D], m1W[:, D]
    z = (X @ W1x.T + m1b).reshape(-1)
    acc, _ = _edge_pass(z, row, col, ew, wcol)
    mean = acc.reshape(N, GNN_H) * inv_deg
    out = jnp.concatenate([X, mean], axis=-1)
    return jax.nn.relu(out @ m2W.T + m2b)


def kernel(x, u, edge_index, edge_w, loc, params):
    xr = x.reshape(-1, TW, FEAT)
    h = _lstm_encode(params['lstm'], xr).reshape(B, CITY, X_EM)
    w = jax.nn.softmax(params['w'], axis=-1)
    locW, locb = params['loc']
    loc_e = loc @ locW.T + locb
    x_loc = jnp.concatenate([h, loc_e], axis=-1)
    g_x = jnp.einsum('cg,bcf->bgf', w, x_loc)
    u_em = jnp.concatenate([params['u1'][u[:, 0]], params['u2'][u[:, 1]],
                            params['u3'][u[:, 2]]], axis=-1)
    eW, eb = params['edge_inf']
    d1 = X_EM + LOC_EM
    E1, E2, E3 = eW[:, :d1], eW[:, d1:2 * d1], eW[:, 2 * d1:]
    gi = jnp.einsum('pg,bgf->bpf', jnp.asarray(S_II), g_x @ E1.T)
    gj = jnp.einsum('pg,bgf->bpf', jnp.asarray(S_JJ), g_x @ E2.T)
    gew = jax.nn.relu(gi + gj + (u_em @ E3.T)[:, None, :] + eb)

    gx = g_x
    for i in range(GNN_LAYER):
        gx = _group_layer(params['group_gnn'][i], gx, gew)
    new_x = jnp.einsum('cg,bgf->bcf', w, gx)
    x0 = jnp.concatenate([h, new_x], axis=-1).reshape(N, X_EM + GNN_H)

    row = edge_index[:, 0].reshape(-1)
    col = edge_index[:, 1].reshape(-1)
    ewf = edge_w.reshape(-1)
    m1W, m1b, m2W, m2b = params['global_gnn'][0]
    D = x0.shape[-1]
    W1x, wcol = m1W[:, :D], m1W[:, D]
    z = (x0 @ W1x.T + m1b).reshape(-1)
    acc, degx = _edge_pass(z, row, col, ewf, wcol)
    deg = degx.reshape(N, 16)[:, :1]
    inv_deg = 1.0 / jnp.maximum(deg, 1.0)
    mean = acc.reshape(N, GNN_H) * inv_deg
    nx = jnp.concatenate([x0, mean], axis=-1)
    nx = jax.nn.relu(nx @ m2W.T + m2b)
    for i in range(1, GNN_LAYER):
        nx = _global_layer(params['global_gnn'][i], nx, row, col, ewf, inv_deg)

    dW, db = params['dec_x_embed']
    wr = params['w']
    dx = (nx @ dW.T + db).reshape(B, CITY, X_EM)
    gx2 = jnp.einsum('cg,bcf->bgf', wr, dx)
    for i in range(GNN_LAYER):
        gx2 = _group_layer(params['dec_group_gnn'][i], gx2, gew)
    nx2 = jnp.einsum('cg,bgf->bcf', wr, gx2)
    nx2 = jnp.concatenate([dx, nx2], axis=-1).reshape(N, X_EM + GNN_H)
    for i in range(GNN_LAYER):
        nx2 = _global_layer(params['dec_global_gnn'][i], nx2, row, col, ewf, inv_deg)
    W1, b1, W2, b2 = params['pred']
    res = jax.nn.relu(jax.nn.relu(nx2 @ W1.T + b1) @ W2.T + b2)
    return res.reshape(-1, CITY, PRED)
